# baseline (speedup 1.0000x reference)
import functools

import jax
import jax.numpy as jnp
import numpy as np
from jax import lax
from jax.experimental import pallas as pl
from jax.experimental.pallas import tpu as pltpu
from jax.experimental.pallas import tpu_sc as plsc

_K = 32
_A = 5
_D = 128
_P = 64
_H = 8
_DK = 16

_TW = 384
_O_K = 0
_O_V = 128
_O_AAF = 256
_O_POS = 320
_O_R = 335
_O_META = 344

_F32 = jnp.float32


def _mm(a, b):
    return jnp.dot(a, b, preferred_element_type=_F32,
                   precision=lax.Precision.HIGHEST)


def _sel(shape, fn):
    i = lax.broadcasted_iota(jnp.int32, shape, 0)
    j = lax.broadcasted_iota(jnp.int32, shape, 1)
    return fn(i, j).astype(_F32)


def _ln_mm(x, s, b):
    w = x.shape[-1]
    mmat = jnp.full((w, w), 1.0 / w, _F32)
    xc = x - _mm(x, mmat)
    v = _mm(xc * xc, mmat)
    return xc * lax.rsqrt(v + 1e-5) * s + b


def _pack_body(feat_ref, aa_ref, pos_ref, meta_ref, waa, lnaas, lnaab,
               lnatts, lnattb, wq, wk, wv, wa15, t_ref, d_ref):
    x = feat_ref[...]
    xn = _ln_mm(x, lnatts[...], lnattb[...])
    q = _mm(xn, wq[...])
    kk = _mm(xn, wk[...])
    v = _mm(xn, wv[...])

    aa = aa_ref[...]
    oh = (lax.broadcasted_iota(jnp.int32, (1, 21), 1).astype(_F32) == aa
          ).astype(_F32)
    aaf = _ln_mm(_mm(oh, waa[...]), lnaas[...], lnaab[...])

    pos = pos_ref[...]
    nat = pos[:, 0:3]
    ca = pos[:, 3:6]
    cc = pos[:, 6:9]

    def norm3(u):
        return u * lax.rsqrt(jnp.sum(u * u, -1, keepdims=True) + 1e-8)

    e1 = norm3(cc - ca)
    u = nat - ca
    e2 = norm3(u - jnp.sum(u * e1, -1, keepdims=True) * e1)

    def cr(i, j):
        return e1[:, i:i + 1] * e2[:, j:j + 1] - e1[:, j:j + 1] * e2[:, i:i + 1]

    r9 = jnp.concatenate([
        e1[:, 0:1], e2[:, 0:1], cr(1, 2),
        e1[:, 1:2], e2[:, 1:2], cr(2, 0),
        e1[:, 2:3], e2[:, 2:3], cr(0, 1)], axis=1)

    y1 = _sel((15, 45), lambda i, j: (j // 9 == i // 3) & ((j % 9) // 3 == i % 3))
    y2 = _sel((9, 45), lambda i, j: j % 9 == i)
    y3 = _sel((45, 15), lambda i, j: (j // 3 == i // 9) & (j % 3 == i % 3))
    pnl15 = _mm(_mm(pos, y1) * _mm(r9, y2), y3)
    csel = _sel((15, 15), lambda i, j: (i >= 3) & (i < 6) & (j % 3 == i - 3))
    canl15 = _mm(pnl15, csel)
    pvbias = -_mm(pnl15, wa15[...])

    meta = meta_ref[...]
    padw = jnp.zeros((x.shape[0], _TW - (_O_META + 4)), _F32)
    t_ref[...] = jnp.concatenate([kk, v, aaf, pos, r9, meta, padw], axis=1)
    pad2 = jnp.zeros((x.shape[0], 18), _F32)
    d_ref[...] = jnp.concatenate(
        [pnl15, canl15, r9, meta, ca, pvbias, q, pad2], axis=1)


def _phase0(featp, aap, posp, metap, p, wa15, n_pad):
    bn = 256
    grid = (n_pad // bn,)
    row = lambda i: (i, 0)
    full = lambda i: (0, 0)
    return pl.pallas_call(
        _pack_body,
        grid=grid,
        in_specs=[
            pl.BlockSpec((bn, _D), row),
            pl.BlockSpec((bn, 1), row),
            pl.BlockSpec((bn, 15), row),
            pl.BlockSpec((bn, 4), row),
            pl.BlockSpec((21, _P), full),
            pl.BlockSpec((1, _P), full),
            pl.BlockSpec((1, _P), full),
            pl.BlockSpec((1, _D), full),
            pl.BlockSpec((1, _D), full),
            pl.BlockSpec((_D, _D), full),
            pl.BlockSpec((_D, _D), full),
            pl.BlockSpec((_D, _D), full),
            pl.BlockSpec((15, _P), full),
        ],
        out_specs=[
            pl.BlockSpec((bn, _TW), row),
            pl.BlockSpec((bn, 256), row),
        ],
        out_shape=[
            jax.ShapeDtypeStruct((n_pad, _TW), _F32),
            jax.ShapeDtypeStruct((n_pad, 256), _F32),
        ],
    )(featp, aap, posp, metap,
      p['w_aa'], p['ln_aa_s'].reshape(1, -1), p['ln_aa_b'].reshape(1, -1),
      p['ln_att_s'].reshape(1, -1), p['ln_att_b'].reshape(1, -1),
      p['wq'], p['wk'], p['wv'], wa15)


def _sc_gather(table, idx):
    rows = idx.shape[0]
    nw = 32
    ch = 128
    per_w = rows // nw
    nch = per_w // ch
    mesh = plsc.VectorSubcoreMesh(core_axis_name="c", subcore_axis_name="s")

    @functools.partial(
        pl.kernel, mesh=mesh,
        out_type=jax.ShapeDtypeStruct((rows, _TW), _F32),
        scratch_types=[
            pltpu.VMEM((ch,), jnp.int32),
            pltpu.VMEM((ch, _TW), _F32),
            pltpu.SemaphoreType.DMA,
        ],
    )
    def gk(table_hbm, idx_hbm, out_hbm, idx_v, rows_v, sem):
        wid = lax.axis_index("s") * 2 + lax.axis_index("c")

        def body(i, carry):
            base = wid * per_w + i * ch
            pltpu.sync_copy(idx_hbm.at[pl.ds(base, ch)], idx_v)
            pltpu.async_copy(table_hbm.at[idx_v], rows_v, sem).wait()
            pltpu.sync_copy(rows_v, out_hbm.at[pl.ds(base, ch)])
            return carry

        lax.fori_loop(0, nch, body, 0)

    return gk(table, idx)


def _edge_body(g_ref, t_ref, f_ref, d_ref, nb_ref,
               w_relpos, w_dist, w_dir, w_rot27, w_b15, lnps, lnpb,
               w_mlp1, b_mlp1, w_mlp2, b_mlp2, wb, wo, lnus, lnub,
               wp1, bp1, wp2, bp2, wu, wg, wr, br, out_ref):
    B = t_ref.shape[0]
    E = B * _K

    g = g_ref[...]
    dn = d_ref[...]

    kg = g[:, _O_K:_O_K + _D]
    vg = g[:, _O_V:_O_V + _D]
    aafg = g[:, _O_AAF:_O_AAF + _P]
    posj = g[:, _O_POS:_O_POS + 15]
    rj = g[:, _O_R:_O_R + 9]
    resj = g[:, _O_META:_O_META + 1]
    chj = g[:, _O_META + 1:_O_META + 2]
    baj = g[:, _O_META + 2:_O_META + 3]
    mj = g[:, _O_META + 3:_O_META + 4]

    repm = _sel((E, B), lambda i, j: i // _K == j)
    redm = _sel((B, E), lambda i, j: j // _K == i)
    dest_e = _mm(repm, dn[:, 15:238])
    canl15 = dest_e[:, 0:15]
    rn = dest_e[:, 15:24]
    resn = dest_e[:, 24:25]
    chn = dest_e[:, 25:26]
    ban = dest_e[:, 26:27]
    mn = dest_e[:, 27:28]
    can = dest_e[:, 28:31]
    pvb = dest_e[:, 31:95]
    qe = dest_e[:, 95:223]

    relp = jnp.clip(resj - resn, -32.0, 32.0) + 32.0
    same = (chj == chn) & (baj == ban)
    idxf = jnp.where(same, relp, 65.0)
    idx66 = _mm(idxf, jnp.ones((1, 66), _F32))
    oh = (lax.broadcasted_iota(jnp.int32, (1, 66), 1).astype(_F32) == idx66
          ).astype(_F32)
    pair = _mm(oh, w_relpos[...])

    caj = posj[:, 3:6]
    dv = caj - can
    d = jnp.sqrt(jnp.sum(dv * dv, -1, keepdims=True) + 1e-8)
    d16 = _mm(d, jnp.ones((1, 16), _F32))
    centers = lax.broadcasted_iota(jnp.int32, (1, 16), 1).astype(_F32) * (22.0 / 15.0)
    sig = 22.0 / 16.0
    rb = jnp.exp(-(((d16 - centers) / sig) ** 2))
    pair += _mm(rb, w_dist[...])

    y1 = _sel((15, 45), lambda i, j: (j // 9 == i // 3) & ((j % 9) // 3 == i % 3))
    y2 = _sel((9, 45), lambda i, j: j % 9 == i)
    y3 = _sel((45, 15), lambda i, j: (j // 3 == i // 9) & (j % 3 == i % 3))
    pjl15 = _mm(_mm(posj, y1) * _mm(rn, y2), y3)

    d15 = pjl15 - canl15
    za = _sel((15, 5), lambda i, j: j == i // 3)
    zb = _sel((5, 15), lambda i, j: j // 3 == i)
    nsq5 = _mm(d15 * d15, za)
    r15 = _mm(lax.rsqrt(nsq5 + 1e-8), zb)
    pair += _mm(d15 * r15, w_dir[...])

    x1 = _sel((9, 27), lambda i, j: (j // 9 == i // 3) & ((j % 9) // 3 == i % 3))
    x2 = _sel((9, 27), lambda i, j: (j // 9 == i // 3) & (j % 3 == i % 3))
    pair += _mm(_mm(rn, x1) * _mm(rj, x2), w_rot27[...])

    pair += _mm(pjl15, w_b15[...]) + pvb

    pair = _ln_mm(pair, lnps[...], lnpb[...])
    pair = pair + aafg
    hmid = jax.nn.gelu(_mm(pair, w_mlp1[...]) + b_mlp1[...])
    pair = _mm(hmid, w_mlp2[...]) + b_mlp2[...]

    selh = _sel((_D, _H), lambda i, j: i // _DK == j)
    exph = _sel((_H, _D), lambda i, j: j // _DK == i)
    qk = _mm(qe * kg, selh) * (1.0 / np.sqrt(_DK))
    logits = qk + _mm(pair, wb[...])
    nbv = nb_ref[...]
    pmf = mn * mj * (nbv != -1.0).astype(_F32)
    logits = jnp.where(pmf > 0.0, logits, -1e9)
    ex = jnp.exp(logits)
    den = _mm(repm, _mm(redm, ex)) + 1e-30
    attn = ex / den
    ae = _mm(attn, exph)
    ov = _mm(redm, ae * vg)
    f1 = f_ref[...] + _mm(ov, wo[...])

    x = _ln_mm(f1, lnus[...], lnub[...])
    lp15 = dn[:, 0:15] - dn[:, 15:30]
    x = x + _mm(jax.nn.gelu(_mm(lp15, wp1[...]) + bp1[...]), wp2[...]) + bp2[...]
    upd = _mm(x, wu[...])
    gate = jax.nn.gelu(_mm(x, wg[...]))
    out_ref[...] = f1 + _mm(gate * upd, wr[...]) + br[...]


def _phase2(G, T, featp, Dn, nbf, p, w_rot27, w_b15, n_pad):
    B = 64
    E = B * _K
    grid = (n_pad // B,)
    erow = lambda i: (i, 0)
    row = lambda i: (i, 0)
    full = lambda i: (0, 0)
    return pl.pallas_call(
        _edge_body,
        grid=grid,
        in_specs=[
            pl.BlockSpec((E, _TW), erow),
            pl.BlockSpec((B, _TW), row),
            pl.BlockSpec((B, _D), row),
            pl.BlockSpec((B, 256), row),
            pl.BlockSpec((E, 1), erow),
            pl.BlockSpec((66, _P), full),
            pl.BlockSpec((16, _P), full),
            pl.BlockSpec((15, _P), full),
            pl.BlockSpec((27, _P), full),
            pl.BlockSpec((15, _P), full),
            pl.BlockSpec((1, _P), full),
            pl.BlockSpec((1, _P), full),
            pl.BlockSpec((_P, 2 * _P), full),
            pl.BlockSpec((1, 2 * _P), full),
            pl.BlockSpec((2 * _P, _P), full),
            pl.BlockSpec((1, _P), full),
            pl.BlockSpec((_P, _H), full),
            pl.BlockSpec((_D, _D), full),
            pl.BlockSpec((1, _D), full),
            pl.BlockSpec((1, _D), full),
            pl.BlockSpec((15, 2 * _D), full),
            pl.BlockSpec((1, 2 * _D), full),
            pl.BlockSpec((2 * _D, _D), full),
            pl.BlockSpec((1, _D), full),
            pl.BlockSpec((_D, 2 * _D), full),
            pl.BlockSpec((_D, 2 * _D), full),
            pl.BlockSpec((2 * _D, _D), full),
            pl.BlockSpec((1, _D), full),
        ],
        out_specs=pl.BlockSpec((B, _D), row),
        out_shape=jax.ShapeDtypeStruct((n_pad, _D), _F32),
    )(G, T, featp, Dn, nbf,
      p['w_relpos'], p['w_dist'], p['w_dir'], w_rot27, w_b15,
      p['ln_pair_s'].reshape(1, -1), p['ln_pair_b'].reshape(1, -1),
      p['w_mlp1'], p['b_mlp1'].reshape(1, -1), p['w_mlp2'],
      p['b_mlp2'].reshape(1, -1), p['wb'], p['wo'],
      p['ln_upd_s'].reshape(1, -1), p['ln_upd_b'].reshape(1, -1),
      p['wp1'], p['bp1'].reshape(1, -1), p['wp2'], p['bp2'].reshape(1, -1),
      p['wu'], p['wg'], p['wr'], p['br'].reshape(1, -1))


def kernel(aa, features, pos, neighbours, resi, chain, batch, mask, params):
    n = features.shape[0]
    n_pad = ((n + 2047) // 2048) * 2048

    def padn(x, fill=0):
        pw = [(0, n_pad - n)] + [(0, 0)] * (x.ndim - 1)
        return jnp.pad(x, pw, constant_values=fill)

    featp = padn(features.astype(_F32))
    aap = padn(aa.astype(_F32)).reshape(n_pad, 1)
    posp = padn(pos.astype(_F32).reshape(n, 15))
    metap = jnp.stack([
        padn(resi.astype(_F32)), padn(chain.astype(_F32)),
        padn(batch.astype(_F32)), padn(mask.astype(_F32)),
    ], axis=1)
    nbp = padn(neighbours)
    nb_idx = jnp.maximum(nbp, 0).astype(jnp.int32).reshape(-1)
    nbf = nbp.astype(_F32).reshape(-1, 1)

    wpv = params['w_pvec'].reshape(_A, _A, 3, _P)
    wa15 = (wpv.sum(1) / 10.0).reshape(3 * _A, _P)
    w_b15 = (wpv.sum(0) / 10.0).reshape(3 * _A, _P)
    w_rot27 = jnp.tile(params['w_rot'], (3, 1))

    T, Dn = _phase0(featp, aap, posp, metap, params, wa15, n_pad)
    G = _sc_gather(T, nb_idx)
    out = _phase2(G, T, featp, Dn, nbf, params, w_rot27, w_b15, n_pad)
    return out[:n]


# speedup vs baseline: 2.7812x; 2.7812x over previous
import functools

import jax
import jax.numpy as jnp
import numpy as np
from jax import lax
from jax.experimental import pallas as pl
from jax.experimental.pallas import tpu as pltpu
from jax.experimental.pallas import tpu_sc as plsc

_K = 32
_A = 5
_D = 128
_P = 64
_H = 8
_DK = 16

_TW = 384
_O_K = 0
_O_V = 128
_O_AAF = 256
_O_POS = 320
_O_R = 335
_O_META = 344

_F32 = jnp.float32


def _mm(a, b):
    return jnp.dot(a, b, preferred_element_type=_F32)


def _sel(shape, fn):
    i = lax.broadcasted_iota(jnp.int32, shape, 0)
    j = lax.broadcasted_iota(jnp.int32, shape, 1)
    return fn(i, j).astype(_F32)


def _ln_mm(x, s, b):
    w = x.shape[-1]
    mmat = jnp.full((w, w), 1.0 / w, _F32)
    xc = x - _mm(x, mmat)
    v = _mm(xc * xc, mmat)
    return xc * lax.rsqrt(v + 1e-5) * s + b


def _pack_body(feat_ref, aa_ref, pos_ref, meta_ref, waa, lnaas, lnaab,
               lnatts, lnattb, wq, wk, wv, wa15, t_ref, d_ref):
    x = feat_ref[...]
    xn = _ln_mm(x, lnatts[...], lnattb[...])
    q = _mm(xn, wq[...])
    kk = _mm(xn, wk[...])
    v = _mm(xn, wv[...])

    aa = aa_ref[...]
    oh = (lax.broadcasted_iota(jnp.int32, (1, 21), 1).astype(_F32) == aa
          ).astype(_F32)
    aaf = _ln_mm(_mm(oh, waa[...]), lnaas[...], lnaab[...])

    pos = pos_ref[...]
    nat = pos[:, 0:3]
    ca = pos[:, 3:6]
    cc = pos[:, 6:9]

    def norm3(u):
        return u * lax.rsqrt(jnp.sum(u * u, -1, keepdims=True) + 1e-8)

    e1 = norm3(cc - ca)
    u = nat - ca
    e2 = norm3(u - jnp.sum(u * e1, -1, keepdims=True) * e1)

    def cr(i, j):
        return e1[:, i:i + 1] * e2[:, j:j + 1] - e1[:, j:j + 1] * e2[:, i:i + 1]

    r9 = jnp.concatenate([
        e1[:, 0:1], e2[:, 0:1], cr(1, 2),
        e1[:, 1:2], e2[:, 1:2], cr(2, 0),
        e1[:, 2:3], e2[:, 2:3], cr(0, 1)], axis=1)

    y1 = _sel((15, 45), lambda i, j: (j // 9 == i // 3) & ((j % 9) // 3 == i % 3))
    y2 = _sel((9, 45), lambda i, j: j % 9 == i)
    y3 = _sel((45, 15), lambda i, j: (j // 3 == i // 9) & (j % 3 == i % 3))
    pnl15 = _mm(_mm(pos, y1) * _mm(r9, y2), y3)
    csel = _sel((15, 15), lambda i, j: (i >= 3) & (i < 6) & (j % 3 == i - 3))
    canl15 = _mm(pnl15, csel)
    pvbias = -_mm(pnl15, wa15[...])

    meta = meta_ref[...]
    padw = jnp.zeros((x.shape[0], _TW - (_O_META + 4)), _F32)
    t_ref[...] = jnp.concatenate([kk, v, aaf, pos, r9, meta, padw], axis=1)
    pad2 = jnp.zeros((x.shape[0], 18), _F32)
    d_ref[...] = jnp.concatenate(
        [pnl15, canl15, r9, meta, ca, pvbias, q, pad2], axis=1)


def _phase0(featp, aap, posp, metap, p, wa15, n_pad):
    bn = 256
    grid = (n_pad // bn,)
    row = lambda i: (i, 0)
    full = lambda i: (0, 0)
    return pl.pallas_call(
        _pack_body,
        grid=grid,
        in_specs=[
            pl.BlockSpec((bn, _D), row),
            pl.BlockSpec((bn, 1), row),
            pl.BlockSpec((bn, 15), row),
            pl.BlockSpec((bn, 4), row),
            pl.BlockSpec((21, _P), full),
            pl.BlockSpec((1, _P), full),
            pl.BlockSpec((1, _P), full),
            pl.BlockSpec((1, _D), full),
            pl.BlockSpec((1, _D), full),
            pl.BlockSpec((_D, _D), full),
            pl.BlockSpec((_D, _D), full),
            pl.BlockSpec((_D, _D), full),
            pl.BlockSpec((15, _P), full),
        ],
        out_specs=[
            pl.BlockSpec((bn, _TW), row),
            pl.BlockSpec((bn, 256), row),
        ],
        out_shape=[
            jax.ShapeDtypeStruct((n_pad, _TW), _F32),
            jax.ShapeDtypeStruct((n_pad, 256), _F32),
        ],
    )(featp, aap, posp, metap,
      p['w_aa'], p['ln_aa_s'].reshape(1, -1), p['ln_aa_b'].reshape(1, -1),
      p['ln_att_s'].reshape(1, -1), p['ln_att_b'].reshape(1, -1),
      p['wq'], p['wk'], p['wv'], wa15)


def _sc_gather(table, idx):
    rows = idx.shape[0]
    nw = 32
    ch = 128
    per_w = rows // nw
    nch = per_w // ch
    mesh = plsc.VectorSubcoreMesh(core_axis_name="c", subcore_axis_name="s")

    @functools.partial(
        pl.kernel, mesh=mesh,
        out_type=jax.ShapeDtypeStruct((rows, _TW), _F32),
        scratch_types=[
            pltpu.VMEM((ch,), jnp.int32),
            pltpu.VMEM((ch,), jnp.int32),
            pltpu.VMEM((ch, _TW), _F32),
            pltpu.VMEM((ch, _TW), _F32),
            pltpu.SemaphoreType.DMA,
            pltpu.SemaphoreType.DMA,
            pltpu.SemaphoreType.DMA,
            pltpu.SemaphoreType.DMA,
        ],
    )
    def gk(table_hbm, idx_hbm, out_hbm, ix0, ix1, buf0, buf1,
           sg0, sg1, sw0, sw1):
        wid = lax.axis_index("s") * 2 + lax.axis_index("c")
        wbase = wid * per_w
        ixs = (ix0, ix1)
        bufs = (buf0, buf1)
        sgs = (sg0, sg1)
        sws = (sw0, sw1)

        def load_idx(i, b):
            pltpu.sync_copy(idx_hbm.at[pl.ds(wbase + i * ch, ch)], ixs[b])

        def start_gather(b):
            pltpu.async_copy(table_hbm.at[ixs[b]], bufs[b], sgs[b])

        def wait_gather(b):
            pltpu.make_async_copy(table_hbm.at[ixs[b]], bufs[b], sgs[b]).wait()

        def start_wb(i, b):
            pltpu.async_copy(bufs[b], out_hbm.at[pl.ds(wbase + i * ch, ch)],
                             sws[b])

        def wait_wb(i, b):
            pltpu.make_async_copy(bufs[b],
                                  out_hbm.at[pl.ds(wbase + i * ch, ch)],
                                  sws[b]).wait()

        load_idx(0, 0)
        start_gather(0)
        load_idx(1, 1)
        start_gather(1)
        wait_gather(0)
        start_wb(0, 0)

        def body(ii, carry):
            i1 = 2 * ii + 1
            wait_wb(i1 - 1, 0)
            load_idx(i1 + 1, 0)
            start_gather(0)
            wait_gather(1)
            start_wb(i1, 1)
            i2 = i1 + 1
            wait_wb(i2 - 1, 1)
            load_idx(i2 + 1, 1)
            start_gather(1)
            wait_gather(0)
            start_wb(i2, 0)
            return carry

        lax.fori_loop(0, (nch - 2) // 2, body, 0)

        wait_gather(1)
        wait_wb(nch - 2, 0)
        start_wb(nch - 1, 1)
        wait_wb(nch - 1, 1)

    return gk(table, idx)


def _edge_body(g_ref, t_ref, f_ref, d_ref, nb_ref,
               w_relpos, w_dist, w_dir, w_rot27, w_b15, lnps, lnpb,
               w_mlp1, b_mlp1, w_mlp2, b_mlp2, wb, wo, lnus, lnub,
               wp1, bp1, wp2, bp2, wu, wg, wr, br, out_ref):
    B = t_ref.shape[0]
    E = B * _K

    g = g_ref[...]
    dn = d_ref[...]

    kg = g[:, _O_K:_O_K + _D]
    vg = g[:, _O_V:_O_V + _D]
    aafg = g[:, _O_AAF:_O_AAF + _P]
    posj = g[:, _O_POS:_O_POS + 15]
    rj = g[:, _O_R:_O_R + 9]
    resj = g[:, _O_META:_O_META + 1]
    chj = g[:, _O_META + 1:_O_META + 2]
    baj = g[:, _O_META + 2:_O_META + 3]
    mj = g[:, _O_META + 3:_O_META + 4]

    repm = _sel((E, B), lambda i, j: i // _K == j)
    redm = _sel((B, E), lambda i, j: j // _K == i)
    dest_e = _mm(repm, dn[:, 15:238])
    canl15 = dest_e[:, 0:15]
    rn = dest_e[:, 15:24]
    resn = dest_e[:, 24:25]
    chn = dest_e[:, 25:26]
    ban = dest_e[:, 26:27]
    mn = dest_e[:, 27:28]
    can = dest_e[:, 28:31]
    pvb = dest_e[:, 31:95]
    qe = dest_e[:, 95:223]

    relp = jnp.clip(resj - resn, -32.0, 32.0) + 32.0
    same = (chj == chn) & (baj == ban)
    idxf = jnp.where(same, relp, 65.0)
    idx66 = _mm(idxf, jnp.ones((1, 66), _F32))
    oh = (lax.broadcasted_iota(jnp.int32, (1, 66), 1).astype(_F32) == idx66
          ).astype(_F32)
    pair = _mm(oh, w_relpos[...])

    caj = posj[:, 3:6]
    dv = caj - can
    d = jnp.sqrt(jnp.sum(dv * dv, -1, keepdims=True) + 1e-8)
    d16 = _mm(d, jnp.ones((1, 16), _F32))
    centers = lax.broadcasted_iota(jnp.int32, (1, 16), 1).astype(_F32) * (22.0 / 15.0)
    sig = 22.0 / 16.0
    rb = jnp.exp(-(((d16 - centers) / sig) ** 2))
    pair += _mm(rb, w_dist[...])

    y1 = _sel((15, 45), lambda i, j: (j // 9 == i // 3) & ((j % 9) // 3 == i % 3))
    y2 = _sel((9, 45), lambda i, j: j % 9 == i)
    y3 = _sel((45, 15), lambda i, j: (j // 3 == i // 9) & (j % 3 == i % 3))
    pjl15 = _mm(_mm(posj, y1) * _mm(rn, y2), y3)

    d15 = pjl15 - canl15
    za = _sel((15, 5), lambda i, j: j == i // 3)
    zb = _sel((5, 15), lambda i, j: j // 3 == i)
    nsq5 = _mm(d15 * d15, za)
    r15 = _mm(lax.rsqrt(nsq5 + 1e-8), zb)
    pair += _mm(d15 * r15, w_dir[...])

    x1 = _sel((9, 27), lambda i, j: (j // 9 == i // 3) & ((j % 9) // 3 == i % 3))
    x2 = _sel((9, 27), lambda i, j: (j // 9 == i // 3) & (j % 3 == i % 3))
    pair += _mm(_mm(rn, x1) * _mm(rj, x2), w_rot27[...])

    pair += _mm(pjl15, w_b15[...]) + pvb

    pair = _ln_mm(pair, lnps[...], lnpb[...])
    pair = pair + aafg
    hmid = jax.nn.gelu(_mm(pair, w_mlp1[...]) + b_mlp1[...])
    pair = _mm(hmid, w_mlp2[...]) + b_mlp2[...]

    selh = _sel((_D, _H), lambda i, j: i // _DK == j)
    exph = _sel((_H, _D), lambda i, j: j // _DK == i)
    qk = _mm(qe * kg, selh) * (1.0 / np.sqrt(_DK))
    logits = qk + _mm(pair, wb[...])
    nbv = nb_ref[...]
    pmf = mn * mj * (nbv != -1.0).astype(_F32)
    logits = jnp.where(pmf > 0.0, logits, -1e9)
    ex = jnp.exp(logits)
    den = _mm(repm, _mm(redm, ex)) + 1e-30
    attn = ex / den
    ae = _mm(attn, exph)
    ov = _mm(redm, ae * vg)
    f1 = f_ref[...] + _mm(ov, wo[...])

    x = _ln_mm(f1, lnus[...], lnub[...])
    lp15 = dn[:, 0:15] - dn[:, 15:30]
    x = x + _mm(jax.nn.gelu(_mm(lp15, wp1[...]) + bp1[...]), wp2[...]) + bp2[...]
    upd = _mm(x, wu[...])
    gate = jax.nn.gelu(_mm(x, wg[...]))
    out_ref[...] = f1 + _mm(gate * upd, wr[...]) + br[...]


def _phase2(G, T, featp, Dn, nbf, p, w_rot27, w_b15, n_pad):
    B = 64
    E = B * _K
    grid = (n_pad // B,)
    erow = lambda i: (i, 0)
    row = lambda i: (i, 0)
    full = lambda i: (0, 0)
    return pl.pallas_call(
        _edge_body,
        grid=grid,
        in_specs=[
            pl.BlockSpec((E, _TW), erow),
            pl.BlockSpec((B, _TW), row),
            pl.BlockSpec((B, _D), row),
            pl.BlockSpec((B, 256), row),
            pl.BlockSpec((E, 1), erow),
            pl.BlockSpec((66, _P), full),
            pl.BlockSpec((16, _P), full),
            pl.BlockSpec((15, _P), full),
            pl.BlockSpec((27, _P), full),
            pl.BlockSpec((15, _P), full),
            pl.BlockSpec((1, _P), full),
            pl.BlockSpec((1, _P), full),
            pl.BlockSpec((_P, 2 * _P), full),
            pl.BlockSpec((1, 2 * _P), full),
            pl.BlockSpec((2 * _P, _P), full),
            pl.BlockSpec((1, _P), full),
            pl.BlockSpec((_P, _H), full),
            pl.BlockSpec((_D, _D), full),
            pl.BlockSpec((1, _D), full),
            pl.BlockSpec((1, _D), full),
            pl.BlockSpec((15, 2 * _D), full),
            pl.BlockSpec((1, 2 * _D), full),
            pl.BlockSpec((2 * _D, _D), full),
            pl.BlockSpec((1, _D), full),
            pl.BlockSpec((_D, 2 * _D), full),
            pl.BlockSpec((_D, 2 * _D), full),
            pl.BlockSpec((2 * _D, _D), full),
            pl.BlockSpec((1, _D), full),
        ],
        out_specs=pl.BlockSpec((B, _D), row),
        out_shape=jax.ShapeDtypeStruct((n_pad, _D), _F32),
    )(G, T, featp, Dn, nbf,
      p['w_relpos'], p['w_dist'], p['w_dir'], w_rot27, w_b15,
      p['ln_pair_s'].reshape(1, -1), p['ln_pair_b'].reshape(1, -1),
      p['w_mlp1'], p['b_mlp1'].reshape(1, -1), p['w_mlp2'],
      p['b_mlp2'].reshape(1, -1), p['wb'], p['wo'],
      p['ln_upd_s'].reshape(1, -1), p['ln_upd_b'].reshape(1, -1),
      p['wp1'], p['bp1'].reshape(1, -1), p['wp2'], p['bp2'].reshape(1, -1),
      p['wu'], p['wg'], p['wr'], p['br'].reshape(1, -1))


def kernel(aa, features, pos, neighbours, resi, chain, batch, mask, params):
    n = features.shape[0]
    n_pad = ((n + 2047) // 2048) * 2048

    def padn(x, fill=0):
        pw = [(0, n_pad - n)] + [(0, 0)] * (x.ndim - 1)
        return jnp.pad(x, pw, constant_values=fill)

    featp = padn(features.astype(_F32))
    aap = padn(aa.astype(_F32)).reshape(n_pad, 1)
    posp = padn(pos.astype(_F32).reshape(n, 15))
    metap = jnp.stack([
        padn(resi.astype(_F32)), padn(chain.astype(_F32)),
        padn(batch.astype(_F32)), padn(mask.astype(_F32)),
    ], axis=1)
    nbp = padn(neighbours)
    nb_idx = jnp.maximum(nbp, 0).astype(jnp.int32).reshape(-1)
    nbf = nbp.astype(_F32).reshape(-1, 1)

    wpv = params['w_pvec'].reshape(_A, _A, 3, _P)
    wa15 = (wpv.sum(1) / 10.0).reshape(3 * _A, _P)
    w_b15 = (wpv.sum(0) / 10.0).reshape(3 * _A, _P)
    w_rot27 = jnp.tile(params['w_rot'], (3, 1))

    T, Dn = _phase0(featp, aap, posp, metap, params, wa15, n_pad)
    G = _sc_gather(T, nb_idx)
    out = _phase2(G, T, featp, Dn, nbf, params, w_rot27, w_b15, n_pad)
    return out[:n]


# speedup vs baseline: 3.1286x; 1.1249x over previous
import functools

import jax
import jax.numpy as jnp
import numpy as np
from jax import lax
from jax.experimental import pallas as pl
from jax.experimental.pallas import tpu as pltpu
from jax.experimental.pallas import tpu_sc as plsc

_K = 32
_A = 5
_D = 128
_P = 64
_H = 8
_DK = 16

_TW = 384
_O_K = 0
_O_V = 128
_O_AAF = 256
_O_POS = 320
_O_R = 335
_O_META = 344

_F32 = jnp.float32


def _mm(a, b):
    return jnp.dot(a, b, preferred_element_type=_F32)


def _sel(shape, fn):
    i = lax.broadcasted_iota(jnp.int32, shape, 0)
    j = lax.broadcasted_iota(jnp.int32, shape, 1)
    return fn(i, j).astype(_F32)


def _ln_mm(x, s, b):
    w = x.shape[-1]
    mmat = jnp.full((w, w), 1.0 / w, _F32)
    xc = x - _mm(x, mmat)
    v = _mm(xc * xc, mmat)
    return xc * lax.rsqrt(v + 1e-5) * s + b


def _pack_body(feat_ref, aa_ref, pos_ref, meta_ref, waa, lnaas, lnaab,
               lnatts, lnattb, wq, wk, wv, wa15, t_ref, d_ref):
    x = feat_ref[...]
    xn = _ln_mm(x, lnatts[...], lnattb[...])
    q = _mm(xn, wq[...])
    kk = _mm(xn, wk[...])
    v = _mm(xn, wv[...])

    aa = aa_ref[...]
    oh = (lax.broadcasted_iota(jnp.int32, (1, 21), 1).astype(_F32) == aa
          ).astype(_F32)
    aaf = _ln_mm(_mm(oh, waa[...]), lnaas[...], lnaab[...])

    pos = pos_ref[...]
    nat = pos[:, 0:3]
    ca = pos[:, 3:6]
    cc = pos[:, 6:9]

    def norm3(u):
        return u * lax.rsqrt(jnp.sum(u * u, -1, keepdims=True) + 1e-8)

    e1 = norm3(cc - ca)
    u = nat - ca
    e2 = norm3(u - jnp.sum(u * e1, -1, keepdims=True) * e1)

    def cr(i, j):
        return e1[:, i:i + 1] * e2[:, j:j + 1] - e1[:, j:j + 1] * e2[:, i:i + 1]

    r9 = jnp.concatenate([
        e1[:, 0:1], e2[:, 0:1], cr(1, 2),
        e1[:, 1:2], e2[:, 1:2], cr(2, 0),
        e1[:, 2:3], e2[:, 2:3], cr(0, 1)], axis=1)

    y1 = _sel((15, 45), lambda i, j: (j // 9 == i // 3) & ((j % 9) // 3 == i % 3))
    y2 = _sel((9, 45), lambda i, j: j % 9 == i)
    y3 = _sel((45, 15), lambda i, j: (j // 3 == i // 9) & (j % 3 == i % 3))
    pnl15 = _mm(_mm(pos, y1) * _mm(r9, y2), y3)
    csel = _sel((15, 15), lambda i, j: (i >= 3) & (i < 6) & (j % 3 == i - 3))
    canl15 = _mm(pnl15, csel)
    pvbias = -_mm(pnl15, wa15[...])

    meta = meta_ref[...]
    padw = jnp.zeros((x.shape[0], _TW - (_O_META + 4)), _F32)
    t_ref[...] = jnp.concatenate([kk, v, aaf, pos, r9, meta, padw], axis=1)
    pad2 = jnp.zeros((x.shape[0], 18), _F32)
    d_ref[...] = jnp.concatenate(
        [pnl15, canl15, r9, meta, ca, pvbias, q, pad2], axis=1)


def _phase0(featp, aap, posp, metap, p, wa15, n_pad):
    bn = 256
    grid = (n_pad // bn,)
    row = lambda i: (i, 0)
    full = lambda i: (0, 0)
    return pl.pallas_call(
        _pack_body,
        grid=grid,
        in_specs=[
            pl.BlockSpec((bn, _D), row),
            pl.BlockSpec((bn, 1), row),
            pl.BlockSpec((bn, 15), row),
            pl.BlockSpec((bn, 4), row),
            pl.BlockSpec((21, _P), full),
            pl.BlockSpec((1, _P), full),
            pl.BlockSpec((1, _P), full),
            pl.BlockSpec((1, _D), full),
            pl.BlockSpec((1, _D), full),
            pl.BlockSpec((_D, _D), full),
            pl.BlockSpec((_D, _D), full),
            pl.BlockSpec((_D, _D), full),
            pl.BlockSpec((15, _P), full),
        ],
        out_specs=[
            pl.BlockSpec((bn, _TW), row),
            pl.BlockSpec((bn, 256), row),
        ],
        out_shape=[
            jax.ShapeDtypeStruct((n_pad, _TW), _F32),
            jax.ShapeDtypeStruct((n_pad, 256), _F32),
        ],
    )(featp, aap, posp, metap,
      p['w_aa'], p['ln_aa_s'].reshape(1, -1), p['ln_aa_b'].reshape(1, -1),
      p['ln_att_s'].reshape(1, -1), p['ln_att_b'].reshape(1, -1),
      p['wq'], p['wk'], p['wv'], wa15)


def _sc_gather(table, idx):
    rows = idx.shape[0]
    nw = 32
    ch = 128
    per_w = rows // nw
    nch = per_w // ch
    mesh = plsc.VectorSubcoreMesh(core_axis_name="c", subcore_axis_name="s")

    @functools.partial(
        pl.kernel, mesh=mesh,
        out_type=jax.ShapeDtypeStruct((rows, _TW), _F32),
        scratch_types=[
            pltpu.VMEM((ch,), jnp.int32),
            pltpu.VMEM((ch,), jnp.int32),
            pltpu.VMEM((ch, _TW), _F32),
            pltpu.VMEM((ch, _TW), _F32),
            pltpu.SemaphoreType.DMA,
            pltpu.SemaphoreType.DMA,
            pltpu.SemaphoreType.DMA,
            pltpu.SemaphoreType.DMA,
        ],
    )
    def gk(table_hbm, idx_hbm, out_hbm, ix0, ix1, buf0, buf1,
           sg0, sg1, sw0, sw1):
        wid = lax.axis_index("s") * 2 + lax.axis_index("c")
        wbase = wid * per_w
        ixs = (ix0, ix1)
        bufs = (buf0, buf1)
        sgs = (sg0, sg1)
        sws = (sw0, sw1)

        def load_idx(i, b):
            pltpu.sync_copy(idx_hbm.at[pl.ds(wbase + i * ch, ch)], ixs[b])

        def start_gather(b):
            pltpu.async_copy(table_hbm.at[ixs[b]], bufs[b], sgs[b])

        def wait_gather(b):
            pltpu.make_async_copy(table_hbm.at[ixs[b]], bufs[b], sgs[b]).wait()

        def start_wb(i, b):
            pltpu.async_copy(bufs[b], out_hbm.at[pl.ds(wbase + i * ch, ch)],
                             sws[b])

        def wait_wb(i, b):
            pltpu.make_async_copy(bufs[b],
                                  out_hbm.at[pl.ds(wbase + i * ch, ch)],
                                  sws[b]).wait()

        load_idx(0, 0)
        start_gather(0)
        load_idx(1, 1)
        start_gather(1)
        wait_gather(0)
        start_wb(0, 0)

        def body(ii, carry):
            i1 = 2 * ii + 1
            wait_wb(i1 - 1, 0)
            load_idx(i1 + 1, 0)
            start_gather(0)
            wait_gather(1)
            start_wb(i1, 1)
            i2 = i1 + 1
            wait_wb(i2 - 1, 1)
            load_idx(i2 + 1, 1)
            start_gather(1)
            wait_gather(0)
            start_wb(i2, 0)
            return carry

        lax.fori_loop(0, (nch - 2) // 2, body, 0)

        wait_gather(1)
        wait_wb(nch - 2, 0)
        start_wb(nch - 1, 1)
        wait_wb(nch - 1, 1)

    return gk(table, idx)


def _edge_body(g_ref, t_ref, f_ref, d_ref, nb_ref,
               w_relpos, w_dist, w_dir, w_rot27, w_b15, lnps, lnpb,
               w_mlp1, b_mlp1, w_mlp2, b_mlp2, wb, wo, lnus, lnub,
               wp1, bp1, wp2, bp2, wu, wg, wr, br, out_ref):
    B = t_ref.shape[0]
    E = B * _K

    g = g_ref[...]
    dn = d_ref[...]

    kg = g[:, _O_K:_O_K + _D]
    vg = g[:, _O_V:_O_V + _D]
    aafg = g[:, _O_AAF:_O_AAF + _P]
    posj = g[:, _O_POS:_O_POS + 15]
    rj = g[:, _O_R:_O_R + 9]
    resj = g[:, _O_META:_O_META + 1]
    chj = g[:, _O_META + 1:_O_META + 2]
    baj = g[:, _O_META + 2:_O_META + 3]
    mj = g[:, _O_META + 3:_O_META + 4]

    repm = _sel((E, B), lambda i, j: i // _K == j)
    redm = _sel((B, E), lambda i, j: j // _K == i)
    dest_e = _mm(repm, dn[:, 15:238])
    canl15 = dest_e[:, 0:15]
    rn = dest_e[:, 15:24]
    resn = dest_e[:, 24:25]
    chn = dest_e[:, 25:26]
    ban = dest_e[:, 26:27]
    mn = dest_e[:, 27:28]
    can = dest_e[:, 28:31]
    pvb = dest_e[:, 31:95]
    qe = dest_e[:, 95:223]

    relp = jnp.clip(resj - resn, -32.0, 32.0) + 32.0
    same = (chj == chn) & (baj == ban)
    idxf = jnp.where(same, relp, 65.0)
    idx66 = _mm(idxf, jnp.ones((1, 66), _F32))
    oh = (lax.broadcasted_iota(jnp.int32, (1, 66), 1).astype(_F32) == idx66
          ).astype(_F32)
    pair = _mm(oh, w_relpos[...])

    caj = posj[:, 3:6]
    dv = caj - can
    d = jnp.sqrt(jnp.sum(dv * dv, -1, keepdims=True) + 1e-8)
    d16 = _mm(d, jnp.ones((1, 16), _F32))
    centers = lax.broadcasted_iota(jnp.int32, (1, 16), 1).astype(_F32) * (22.0 / 15.0)
    sig = 22.0 / 16.0
    rb = jnp.exp(-(((d16 - centers) / sig) ** 2))
    pair += _mm(rb, w_dist[...])

    y1 = _sel((15, 45), lambda i, j: (j // 9 == i // 3) & ((j % 9) // 3 == i % 3))
    y2 = _sel((9, 45), lambda i, j: j % 9 == i)
    y3 = _sel((45, 15), lambda i, j: (j // 3 == i // 9) & (j % 3 == i % 3))
    pjl15 = _mm(_mm(posj, y1) * _mm(rn, y2), y3)

    d15 = pjl15 - canl15
    za = _sel((15, 5), lambda i, j: j == i // 3)
    zb = _sel((5, 15), lambda i, j: j // 3 == i)
    nsq5 = _mm(d15 * d15, za)
    r15 = _mm(lax.rsqrt(nsq5 + 1e-8), zb)
    pair += _mm(d15 * r15, w_dir[...])

    x1 = _sel((9, 27), lambda i, j: (j // 9 == i // 3) & ((j % 9) // 3 == i % 3))
    x2 = _sel((9, 27), lambda i, j: (j // 9 == i // 3) & (j % 3 == i % 3))
    pair += _mm(_mm(rn, x1) * _mm(rj, x2), w_rot27[...])

    pair += _mm(pjl15, w_b15[...]) + pvb

    pair = _ln_mm(pair, lnps[...], lnpb[...])
    pair = pair + aafg
    hmid = jax.nn.gelu(_mm(pair, w_mlp1[...]) + b_mlp1[...])
    pair = _mm(hmid, w_mlp2[...]) + b_mlp2[...]

    selh = _sel((_D, _H), lambda i, j: i // _DK == j)
    exph = _sel((_H, _D), lambda i, j: j // _DK == i)
    qk = _mm(qe * kg, selh) * (1.0 / np.sqrt(_DK))
    logits = qk + _mm(pair, wb[...])
    nbv = nb_ref[...]
    pmf = mn * mj * (nbv != -1.0).astype(_F32)
    logits = jnp.where(pmf > 0.0, logits, -1e9)
    ex = jnp.exp(logits)
    den = _mm(repm, _mm(redm, ex)) + 1e-30
    attn = ex / den
    ae = _mm(attn, exph)
    ov = _mm(redm, ae * vg)
    f1 = f_ref[...] + _mm(ov, wo[...])

    x = _ln_mm(f1, lnus[...], lnub[...])
    lp15 = dn[:, 0:15] - dn[:, 15:30]
    x = x + _mm(jax.nn.gelu(_mm(lp15, wp1[...]) + bp1[...]), wp2[...]) + bp2[...]
    upd = _mm(x, wu[...])
    gate = jax.nn.gelu(_mm(x, wg[...]))
    out_ref[...] = f1 + _mm(gate * upd, wr[...]) + br[...]


def _phase2(G, T, featp, Dn, nbf, p, w_rot27, w_b15, n_pad):
    B = 64
    E = B * _K
    grid = (n_pad // B,)
    erow = lambda i: (i, 0)
    row = lambda i: (i, 0)
    full = lambda i: (0, 0)
    return pl.pallas_call(
        _edge_body,
        grid=grid,
        in_specs=[
            pl.BlockSpec((E, _TW), erow),
            pl.BlockSpec((B, _TW), row),
            pl.BlockSpec((B, _D), row),
            pl.BlockSpec((B, 256), row),
            pl.BlockSpec((E, 1), erow),
            pl.BlockSpec((66, _P), full),
            pl.BlockSpec((16, _P), full),
            pl.BlockSpec((15, _P), full),
            pl.BlockSpec((27, _P), full),
            pl.BlockSpec((15, _P), full),
            pl.BlockSpec((1, _P), full),
            pl.BlockSpec((1, _P), full),
            pl.BlockSpec((_P, 2 * _P), full),
            pl.BlockSpec((1, 2 * _P), full),
            pl.BlockSpec((2 * _P, _P), full),
            pl.BlockSpec((1, _P), full),
            pl.BlockSpec((_P, _H), full),
            pl.BlockSpec((_D, _D), full),
            pl.BlockSpec((1, _D), full),
            pl.BlockSpec((1, _D), full),
            pl.BlockSpec((15, 2 * _D), full),
            pl.BlockSpec((1, 2 * _D), full),
            pl.BlockSpec((2 * _D, _D), full),
            pl.BlockSpec((1, _D), full),
            pl.BlockSpec((_D, 2 * _D), full),
            pl.BlockSpec((_D, 2 * _D), full),
            pl.BlockSpec((2 * _D, _D), full),
            pl.BlockSpec((1, _D), full),
        ],
        out_specs=pl.BlockSpec((B, _D), row),
        out_shape=jax.ShapeDtypeStruct((n_pad, _D), _F32),
    )(G, T, featp, Dn, nbf,
      p['w_relpos'], p['w_dist'], p['w_dir'], w_rot27, w_b15,
      p['ln_pair_s'].reshape(1, -1), p['ln_pair_b'].reshape(1, -1),
      p['w_mlp1'], p['b_mlp1'].reshape(1, -1), p['w_mlp2'],
      p['b_mlp2'].reshape(1, -1), p['wb'], p['wo'],
      p['ln_upd_s'].reshape(1, -1), p['ln_upd_b'].reshape(1, -1),
      p['wp1'], p['bp1'].reshape(1, -1), p['wp2'], p['bp2'].reshape(1, -1),
      p['wu'], p['wg'], p['wr'], p['br'].reshape(1, -1))


def kernel(aa, features, pos, neighbours, resi, chain, batch, mask, params):
    n = features.shape[0]
    n_pad = ((n + 2047) // 2048) * 2048

    def padn(x, fill=0):
        pw = [(0, n_pad - n)] + [(0, 0)] * (x.ndim - 1)
        return jnp.pad(x, pw, constant_values=fill)

    featp = padn(features.astype(_F32))
    aap = padn(aa.astype(_F32)).reshape(n_pad, 1)
    posp = padn(pos.astype(_F32).reshape(n, 15))
    metap = jnp.stack([
        padn(resi.astype(_F32)), padn(chain.astype(_F32)),
        padn(batch.astype(_F32)), padn(mask.astype(_F32)),
    ], axis=1)
    nbp = padn(neighbours)
    nb_idx = jnp.maximum(nbp, 0).astype(jnp.int32).reshape(-1)
    nbf = nbp.astype(_F32).reshape(-1, 1)

    wpv = params['w_pvec'].reshape(_A, _A, 3, _P)
    wa15 = (wpv.sum(1) / 10.0).reshape(3 * _A, _P)
    w_b15 = (wpv.sum(0) / 10.0).reshape(3 * _A, _P)
    w_rot27 = jnp.tile(params['w_rot'], (3, 1))

    T, Dn = _phase0(featp, aap, posp, metap, params, wa15, n_pad)

    nchunks = 4
    nc = n_pad // nchunks
    outs = []
    for c in range(nchunks):
        Gc = _sc_gather(T, lax.dynamic_slice_in_dim(nb_idx, c * nc * _K,
                                                    nc * _K))
        outs.append(_phase2(
            Gc,
            lax.dynamic_slice_in_dim(T, c * nc, nc),
            lax.dynamic_slice_in_dim(featp, c * nc, nc),
            lax.dynamic_slice_in_dim(Dn, c * nc, nc),
            lax.dynamic_slice_in_dim(nbf, c * nc * _K, nc * _K),
            params, w_rot27, w_b15, nc))
    out = jnp.concatenate(outs, axis=0)
    return out[:n]


# speedup vs baseline: 3.1815x; 1.0169x over previous
import functools

import jax
import jax.numpy as jnp
import numpy as np
from jax import lax
from jax.experimental import pallas as pl
from jax.experimental.pallas import tpu as pltpu
from jax.experimental.pallas import tpu_sc as plsc

_K = 32
_A = 5
_D = 128
_P = 64
_H = 8
_DK = 16

_TW = 384
_O_K = 0
_O_V = 128
_O_AAF = 256
_O_POS = 320
_O_R = 335
_O_META = 344

_F32 = jnp.float32


def _mm(a, b):
    return jnp.dot(a, b, preferred_element_type=_F32)


def _sel(shape, fn):
    i = lax.broadcasted_iota(jnp.int32, shape, 0)
    j = lax.broadcasted_iota(jnp.int32, shape, 1)
    return fn(i, j).astype(_F32)


def _ln_mm(x, s, b):
    w = x.shape[-1]
    mmat = jnp.full((w, w), 1.0 / w, _F32)
    xc = x - _mm(x, mmat)
    v = _mm(xc * xc, mmat)
    return xc * lax.rsqrt(v + 1e-5) * s + b


def _pack_body(feat_ref, aa_ref, pos_ref, meta_ref, waa, lnaas, lnaab,
               lnatts, lnattb, wq, wk, wv, wa15, t_ref, d_ref):
    x = feat_ref[...]
    xn = _ln_mm(x, lnatts[...], lnattb[...])
    q = _mm(xn, wq[...])
    kk = _mm(xn, wk[...])
    v = _mm(xn, wv[...])

    aa = aa_ref[...]
    oh = (lax.broadcasted_iota(jnp.int32, (1, 21), 1).astype(_F32) == aa
          ).astype(_F32)
    aaf = _ln_mm(_mm(oh, waa[...]), lnaas[...], lnaab[...])

    pos = pos_ref[...]
    nat = pos[:, 0:3]
    ca = pos[:, 3:6]
    cc = pos[:, 6:9]

    def norm3(u):
        return u * lax.rsqrt(jnp.sum(u * u, -1, keepdims=True) + 1e-8)

    e1 = norm3(cc - ca)
    u = nat - ca
    e2 = norm3(u - jnp.sum(u * e1, -1, keepdims=True) * e1)

    def cr(i, j):
        return e1[:, i:i + 1] * e2[:, j:j + 1] - e1[:, j:j + 1] * e2[:, i:i + 1]

    r9 = jnp.concatenate([
        e1[:, 0:1], e2[:, 0:1], cr(1, 2),
        e1[:, 1:2], e2[:, 1:2], cr(2, 0),
        e1[:, 2:3], e2[:, 2:3], cr(0, 1)], axis=1)

    y1 = _sel((15, 45), lambda i, j: (j // 9 == i // 3) & ((j % 9) // 3 == i % 3))
    y2 = _sel((9, 45), lambda i, j: j % 9 == i)
    y3 = _sel((45, 15), lambda i, j: (j // 3 == i // 9) & (j % 3 == i % 3))
    pnl15 = _mm(_mm(pos, y1) * _mm(r9, y2), y3)
    csel = _sel((15, 15), lambda i, j: (i >= 3) & (i < 6) & (j % 3 == i - 3))
    canl15 = _mm(pnl15, csel)
    pvbias = -_mm(pnl15, wa15[...])

    meta = meta_ref[...]
    padw = jnp.zeros((x.shape[0], _TW - (_O_META + 4)), _F32)
    t_ref[...] = jnp.concatenate([kk, v, aaf, pos, r9, meta, padw], axis=1)
    pad2 = jnp.zeros((x.shape[0], 18), _F32)
    d_ref[...] = jnp.concatenate(
        [pnl15, canl15, r9, meta, ca, pvbias, q, pad2], axis=1)


def _phase0(featp, aap, posp, metap, p, wa15, n_pad):
    bn = 256
    grid = (n_pad // bn,)
    row = lambda i: (i, 0)
    full = lambda i: (0, 0)
    return pl.pallas_call(
        _pack_body,
        grid=grid,
        in_specs=[
            pl.BlockSpec((bn, _D), row),
            pl.BlockSpec((bn, 1), row),
            pl.BlockSpec((bn, 15), row),
            pl.BlockSpec((bn, 4), row),
            pl.BlockSpec((21, _P), full),
            pl.BlockSpec((1, _P), full),
            pl.BlockSpec((1, _P), full),
            pl.BlockSpec((1, _D), full),
            pl.BlockSpec((1, _D), full),
            pl.BlockSpec((_D, _D), full),
            pl.BlockSpec((_D, _D), full),
            pl.BlockSpec((_D, _D), full),
            pl.BlockSpec((15, _P), full),
        ],
        out_specs=[
            pl.BlockSpec((bn, _TW), row),
            pl.BlockSpec((bn, 256), row),
        ],
        out_shape=[
            jax.ShapeDtypeStruct((n_pad, _TW), _F32),
            jax.ShapeDtypeStruct((n_pad, 256), _F32),
        ],
    )(featp, aap, posp, metap,
      p['w_aa'], p['ln_aa_s'].reshape(1, -1), p['ln_aa_b'].reshape(1, -1),
      p['ln_att_s'].reshape(1, -1), p['ln_att_b'].reshape(1, -1),
      p['wq'], p['wk'], p['wv'], wa15)


def _sc_gather(table, idx):
    rows = idx.shape[0]
    nw = 32
    ch = 128
    per_w = rows // nw
    nch = per_w // ch
    mesh = plsc.VectorSubcoreMesh(core_axis_name="c", subcore_axis_name="s")

    @functools.partial(
        pl.kernel, mesh=mesh,
        out_type=jax.ShapeDtypeStruct((rows, _TW), _F32),
        scratch_types=[
            pltpu.VMEM((ch,), jnp.int32),
            pltpu.VMEM((ch,), jnp.int32),
            pltpu.VMEM((ch, _TW), _F32),
            pltpu.VMEM((ch, _TW), _F32),
            pltpu.SemaphoreType.DMA,
            pltpu.SemaphoreType.DMA,
            pltpu.SemaphoreType.DMA,
            pltpu.SemaphoreType.DMA,
        ],
    )
    def gk(table_hbm, idx_hbm, out_hbm, ix0, ix1, buf0, buf1,
           sg0, sg1, sw0, sw1):
        wid = lax.axis_index("s") * 2 + lax.axis_index("c")
        wbase = wid * per_w
        ixs = (ix0, ix1)
        bufs = (buf0, buf1)
        sgs = (sg0, sg1)
        sws = (sw0, sw1)

        def load_idx(i, b):
            pltpu.sync_copy(idx_hbm.at[pl.ds(wbase + i * ch, ch)], ixs[b])

        def start_gather(b):
            pltpu.async_copy(table_hbm.at[ixs[b]], bufs[b], sgs[b])

        def wait_gather(b):
            pltpu.make_async_copy(table_hbm.at[ixs[b]], bufs[b], sgs[b]).wait()

        def start_wb(i, b):
            pltpu.async_copy(bufs[b], out_hbm.at[pl.ds(wbase + i * ch, ch)],
                             sws[b])

        def wait_wb(i, b):
            pltpu.make_async_copy(bufs[b],
                                  out_hbm.at[pl.ds(wbase + i * ch, ch)],
                                  sws[b]).wait()

        load_idx(0, 0)
        start_gather(0)
        load_idx(1, 1)
        start_gather(1)
        wait_gather(0)
        start_wb(0, 0)

        def body(ii, carry):
            i1 = 2 * ii + 1
            wait_wb(i1 - 1, 0)
            load_idx(i1 + 1, 0)
            start_gather(0)
            wait_gather(1)
            start_wb(i1, 1)
            i2 = i1 + 1
            wait_wb(i2 - 1, 1)
            load_idx(i2 + 1, 1)
            start_gather(1)
            wait_gather(0)
            start_wb(i2, 0)
            return carry

        lax.fori_loop(0, (nch - 2) // 2, body, 0)

        wait_gather(1)
        wait_wb(nch - 2, 0)
        start_wb(nch - 1, 1)
        wait_wb(nch - 1, 1)

    return gk(table, idx)


def _edge_body(g_ref, t_ref, f_ref, d_ref, nb_ref,
               w_relpos, w_dist, w_dir, w_rot27, w_b15, lnps, lnpb,
               w_mlp1, b_mlp1, w_mlp2, b_mlp2, wb, wo, lnus, lnub,
               wp1, bp1, wp2, bp2, wu, wg, wr, br, out_ref):
    B = t_ref.shape[0]
    E = B * _K

    g = g_ref[...]
    dn = d_ref[...]

    kg = g[:, _O_K:_O_K + _D]
    vg = g[:, _O_V:_O_V + _D]
    aafg = g[:, _O_AAF:_O_AAF + _P]
    posj = g[:, _O_POS:_O_POS + 15]
    rj = g[:, _O_R:_O_R + 9]
    resj = g[:, _O_META:_O_META + 1]
    chj = g[:, _O_META + 1:_O_META + 2]
    baj = g[:, _O_META + 2:_O_META + 3]
    mj = g[:, _O_META + 3:_O_META + 4]

    repm = _sel((E, B), lambda i, j: i // _K == j)
    redm = _sel((B, E), lambda i, j: j // _K == i)
    dest_e = _mm(repm, dn[:, 15:238])
    canl15 = dest_e[:, 0:15]
    rn = dest_e[:, 15:24]
    resn = dest_e[:, 24:25]
    chn = dest_e[:, 25:26]
    ban = dest_e[:, 26:27]
    mn = dest_e[:, 27:28]
    can = dest_e[:, 28:31]
    pvb = dest_e[:, 31:95]
    qe = dest_e[:, 95:223]

    relp = jnp.clip(resj - resn, -32.0, 32.0) + 32.0
    same = (chj == chn) & (baj == ban)
    idxf = jnp.where(same, relp, 65.0)
    idx66 = _mm(idxf, jnp.ones((1, 66), _F32))
    oh = (lax.broadcasted_iota(jnp.int32, (1, 66), 1).astype(_F32) == idx66
          ).astype(_F32)
    pair = _mm(oh, w_relpos[...])

    caj = posj[:, 3:6]
    dv = caj - can
    d = jnp.sqrt(jnp.sum(dv * dv, -1, keepdims=True) + 1e-8)
    d16 = _mm(d, jnp.ones((1, 16), _F32))
    centers = lax.broadcasted_iota(jnp.int32, (1, 16), 1).astype(_F32) * (22.0 / 15.0)
    sig = 22.0 / 16.0
    rb = jnp.exp(-(((d16 - centers) / sig) ** 2))
    pair += _mm(rb, w_dist[...])

    y1 = _sel((15, 45), lambda i, j: (j // 9 == i // 3) & ((j % 9) // 3 == i % 3))
    y2 = _sel((9, 45), lambda i, j: j % 9 == i)
    y3 = _sel((45, 15), lambda i, j: (j // 3 == i // 9) & (j % 3 == i % 3))
    pjl15 = _mm(_mm(posj, y1) * _mm(rn, y2), y3)

    d15 = pjl15 - canl15
    za = _sel((15, 5), lambda i, j: j == i // 3)
    zb = _sel((5, 15), lambda i, j: j // 3 == i)
    nsq5 = _mm(d15 * d15, za)
    r15 = _mm(lax.rsqrt(nsq5 + 1e-8), zb)
    pair += _mm(d15 * r15, w_dir[...])

    x1 = _sel((9, 27), lambda i, j: (j // 9 == i // 3) & ((j % 9) // 3 == i % 3))
    x2 = _sel((9, 27), lambda i, j: (j // 9 == i // 3) & (j % 3 == i % 3))
    pair += _mm(_mm(rn, x1) * _mm(rj, x2), w_rot27[...])

    pair += _mm(pjl15, w_b15[...]) + pvb

    pair = _ln_mm(pair, lnps[...], lnpb[...])
    pair = pair + aafg
    hmid = jax.nn.gelu(_mm(pair, w_mlp1[...]) + b_mlp1[...])
    pair = _mm(hmid, w_mlp2[...]) + b_mlp2[...]

    selh = _sel((_D, _H), lambda i, j: i // _DK == j)
    exph = _sel((_H, _D), lambda i, j: j // _DK == i)
    qk = _mm(qe * kg, selh) * (1.0 / np.sqrt(_DK))
    logits = qk + _mm(pair, wb[...])
    nbv = nb_ref[...]
    pmf = mn * mj * (nbv != -1.0).astype(_F32)
    logits = jnp.where(pmf > 0.0, logits, -1e9)
    ex = jnp.exp(logits)
    den = _mm(repm, _mm(redm, ex)) + 1e-30
    attn = ex / den
    ae = _mm(attn, exph)
    ov = _mm(redm, ae * vg)
    f1 = f_ref[...] + _mm(ov, wo[...])

    x = _ln_mm(f1, lnus[...], lnub[...])
    lp15 = dn[:, 0:15] - dn[:, 15:30]
    x = x + _mm(jax.nn.gelu(_mm(lp15, wp1[...]) + bp1[...]), wp2[...]) + bp2[...]
    upd = _mm(x, wu[...])
    gate = jax.nn.gelu(_mm(x, wg[...]))
    out_ref[...] = f1 + _mm(gate * upd, wr[...]) + br[...]


def _phase2(G, T, featp, Dn, nbf, p, w_rot27, w_b15, n_pad):
    B = 64
    E = B * _K
    grid = (n_pad // B,)
    erow = lambda i: (i, 0)
    row = lambda i: (i, 0)
    full = lambda i: (0, 0)
    return pl.pallas_call(
        _edge_body,
        grid=grid,
        in_specs=[
            pl.BlockSpec((E, _TW), erow),
            pl.BlockSpec((B, _TW), row),
            pl.BlockSpec((B, _D), row),
            pl.BlockSpec((B, 256), row),
            pl.BlockSpec((E, 1), erow),
            pl.BlockSpec((66, _P), full),
            pl.BlockSpec((16, _P), full),
            pl.BlockSpec((15, _P), full),
            pl.BlockSpec((27, _P), full),
            pl.BlockSpec((15, _P), full),
            pl.BlockSpec((1, _P), full),
            pl.BlockSpec((1, _P), full),
            pl.BlockSpec((_P, 2 * _P), full),
            pl.BlockSpec((1, 2 * _P), full),
            pl.BlockSpec((2 * _P, _P), full),
            pl.BlockSpec((1, _P), full),
            pl.BlockSpec((_P, _H), full),
            pl.BlockSpec((_D, _D), full),
            pl.BlockSpec((1, _D), full),
            pl.BlockSpec((1, _D), full),
            pl.BlockSpec((15, 2 * _D), full),
            pl.BlockSpec((1, 2 * _D), full),
            pl.BlockSpec((2 * _D, _D), full),
            pl.BlockSpec((1, _D), full),
            pl.BlockSpec((_D, 2 * _D), full),
            pl.BlockSpec((_D, 2 * _D), full),
            pl.BlockSpec((2 * _D, _D), full),
            pl.BlockSpec((1, _D), full),
        ],
        out_specs=pl.BlockSpec((B, _D), row),
        out_shape=jax.ShapeDtypeStruct((n_pad, _D), _F32),
    )(G, T, featp, Dn, nbf,
      p['w_relpos'], p['w_dist'], p['w_dir'], w_rot27, w_b15,
      p['ln_pair_s'].reshape(1, -1), p['ln_pair_b'].reshape(1, -1),
      p['w_mlp1'], p['b_mlp1'].reshape(1, -1), p['w_mlp2'],
      p['b_mlp2'].reshape(1, -1), p['wb'], p['wo'],
      p['ln_upd_s'].reshape(1, -1), p['ln_upd_b'].reshape(1, -1),
      p['wp1'], p['bp1'].reshape(1, -1), p['wp2'], p['bp2'].reshape(1, -1),
      p['wu'], p['wg'], p['wr'], p['br'].reshape(1, -1))


def kernel(aa, features, pos, neighbours, resi, chain, batch, mask, params):
    n = features.shape[0]
    n_pad = ((n + 2047) // 2048) * 2048

    def padn(x, fill=0):
        pw = [(0, n_pad - n)] + [(0, 0)] * (x.ndim - 1)
        return jnp.pad(x, pw, constant_values=fill)

    featp = padn(features.astype(_F32))
    aap = padn(aa.astype(_F32)).reshape(n_pad, 1)
    posp = padn(pos.astype(_F32).reshape(n, 15))
    metap = jnp.stack([
        padn(resi.astype(_F32)), padn(chain.astype(_F32)),
        padn(batch.astype(_F32)), padn(mask.astype(_F32)),
    ], axis=1)
    nbp = padn(neighbours)
    nb_idx = jnp.maximum(nbp, 0).astype(jnp.int32).reshape(-1)
    nbf = nbp.astype(_F32).reshape(-1, 1)

    wpv = params['w_pvec'].reshape(_A, _A, 3, _P)
    wa15 = (wpv.sum(1) / 10.0).reshape(3 * _A, _P)
    w_b15 = (wpv.sum(0) / 10.0).reshape(3 * _A, _P)
    w_rot27 = jnp.tile(params['w_rot'], (3, 1))

    T, Dn = _phase0(featp, aap, posp, metap, params, wa15, n_pad)

    nchunks = 8
    nc = n_pad // nchunks
    outs = []
    for c in range(nchunks):
        Gc = _sc_gather(T, lax.dynamic_slice_in_dim(nb_idx, c * nc * _K,
                                                    nc * _K))
        outs.append(_phase2(
            Gc,
            lax.dynamic_slice_in_dim(T, c * nc, nc),
            lax.dynamic_slice_in_dim(featp, c * nc, nc),
            lax.dynamic_slice_in_dim(Dn, c * nc, nc),
            lax.dynamic_slice_in_dim(nbf, c * nc * _K, nc * _K),
            params, w_rot27, w_b15, nc))
    out = jnp.concatenate(outs, axis=0)
    return out[:n]
